# baseline (speedup 1.0000x reference)
import functools

import jax
import jax.numpy as jnp
from jax import lax
from jax.experimental import pallas as pl
from jax.experimental.pallas import tpu as pltpu
from jax.experimental.pallas import tpu_sc as plsc

N = 4194304
R = 65536
NC = 2
NS = 16
W = NC * NS
C = N // W
CH = 2048
NCH = C // CH
NBT = CH // 128 + 1
CP = CH + 128
NB = 4
RT = R // NS


def _vgather(x, idx):
    dnums = lax.GatherDimensionNumbers(
        offset_dims=(), collapsed_slice_dims=(0,), start_index_map=(0,))
    return lax.gather(x, idx[:, None], dnums, (1,),
                      mode=lax.GatherScatterMode.PROMISE_IN_BOUNDS)


def _sc_body(ids_hbm, r_hbm, g_hbm, b_hbm, w_hbm, zeros_hbm, out_hbm,
             ids_v, r_v, g_v, b_v, w_v, idc_v, cr_v, cg_v, cb_v,
             acc_r, acc_g, acc_b, np_s, sem_in, sem_sc, sem_z):
    cid = lax.axis_index("c")
    sid = lax.axis_index("s")
    wid = cid * NS + sid

    for acc in (acc_r, acc_g, acc_b):
        pltpu.async_copy(zeros_hbm.at[pl.ds(sid * RT, RT)],
                         acc.at[pl.ds(sid * RT, RT)], sem_z).wait()
    for p in range(NB):
        pltpu.async_copy(ids_hbm.at[pl.ds(0, CP)], idc_v[p], sem_z).wait()
    plsc.subcore_barrier()

    planes = ((r_hbm, r_v, cr_v, acc_r),
              (g_hbm, g_v, cg_v, acc_g),
              (b_hbm, b_v, cb_v, acc_b))

    ii = lax.iota(jnp.int32, 16)
    nxt = jnp.minimum(ii + 1, 15)
    prv = jnp.maximum(ii - 1, 0)
    last = ii == 15
    first = ii == 0
    zeros16 = jnp.zeros((16,), jnp.float32)
    neg1 = jnp.full((16,), -1, jnp.int32)

    def in_copies(p, ch):
        base = wid * C + ch * CH
        copies = [pltpu.make_async_copy(ids_hbm.at[pl.ds(base, CH)],
                                        ids_v[p].at[pl.ds(0, CH)],
                                        sem_in[p]),
                  pltpu.make_async_copy(w_hbm.at[pl.ds(base, CH)],
                                        w_v[p], sem_in[p])]
        copies.extend(
            pltpu.make_async_copy(x_hbm.at[pl.ds(base, CH)],
                                  x_v[p], sem_in[p])
            for x_hbm, x_v, _, _ in planes)
        return copies

    def issue_in(p, ch):
        for c in in_copies(p, ch):
            c.start()

    def wait_in(p, ch):
        for c in in_copies(p, ch):
            c.wait()

    def compute(p):
        def group(g4, ptr):
            for u in range(4):
                o = 64 * g4 + 16 * u
                id16 = ids_v[p][pl.ds(o, 16)]
                w16 = w_v[p][pl.ds(o, 16)]
                ends = (id16 != ids_v[p][pl.ds(o + 1, 16)]) | last
                endpos = plsc.cummax(jnp.where(ends, ii, neg1))
                pe = jnp.where(first, neg1, _vgather(endpos, prv))
                pec = jnp.maximum(pe, 0)
                no_pe = pe < 0
                pos = ptr + plsc.cumsum(ends.astype(jnp.int32)) - 1
                plsc.store_scatter(idc_v[p], [pos], id16, mask=ends)
                for _, x_v, c_v, _ in planes:
                    s16 = plsc.cumsum(x_v[p][pl.ds(o, 16)] * w16)
                    d16 = s16 - jnp.where(no_pe, zeros16, _vgather(s16, pec))
                    plsc.store_scatter(c_v[p], [pos], d16, mask=ends)
                ptr = ptr + plsc.all_reduce_population_count(ends)
            return ptr

        ptrv = lax.fori_loop(0, CH // 64, group,
                             jnp.zeros((16,), jnp.int32))
        return ptrv[0]

    def zero_tail(p, ptr):
        pb = lax.shift_left(lax.shift_right_logical(ptr, 4), 4)
        msk = ii >= (ptr - pb)
        khi = lax.shift_left(lax.shift_right_logical(ptr + 127, 7), 3)

        for _, _, c_v, _ in planes:
            plsc.store_scatter(c_v[p], [pb + ii], zeros16, mask=msk)

            def zloop(k, _):
                c_v[p][pl.ds(16 * k, 16)] = zeros16
                return 0

            lax.fori_loop(lax.shift_right_logical(ptr + 15, 4), khi, zloop, 0)

    def fire_scatters(p, ptr):
        np_s[p] = ptr

        def fire(b, _):
            for _, _, c_v, acc in planes:
                pltpu.async_copy(
                    c_v[p].at[pl.ds(128 * b, 128)],
                    acc.at[idc_v[p].at[pl.ds(128 * b, 128)]],
                    sem_sc[p], add=True)
            return 0

        lax.fori_loop(0, lax.shift_right_logical(ptr + 127, 7), fire, 0)

    def drain_scatters(p):
        def drain(b, _):
            for _ in range(3):
                pltpu.make_async_copy(
                    w_hbm.at[pl.ds(0, 128)],
                    w_v[p].at[pl.ds(0, 128)], sem_sc[p]).wait()
            return 0

        lax.fori_loop(0, lax.shift_right_logical(np_s[p] + 127, 7), drain, 0)

    def step(pp, ch):
        wait_in(pp, ch)
        ptr = compute(pp)
        zero_tail(pp, ptr)
        fire_scatters(pp, ptr)

    issue_in(0, 0)
    issue_in(1, 1)

    def quad(t, _):
        for pp in range(NB):
            ch = NB * t + pp
            qq = (pp + 2) % NB

            def prefetch():
                issue_in(qq, ch + 2)

            def drain_and_prefetch():
                drain_scatters(qq)
                prefetch()

            if pp < 2:
                lax.cond(t > 0, drain_and_prefetch, prefetch)
            else:
                drain_scatters(qq)
                lax.cond(ch < NCH - 2, prefetch, lambda: None)
            step(pp, ch)
        return 0

    lax.fori_loop(0, NCH // NB, quad, 0)
    drain_scatters((NCH - 2) % NB)
    drain_scatters((NCH - 1) % NB)
    plsc.subcore_barrier()

    for x, (_, _, _, acc) in enumerate(planes):
        pltpu.async_copy(acc.at[pl.ds(sid * RT, RT)],
                         out_hbm.at[cid].at[pl.ds(x * R + sid * RT, RT)],
                         sem_z).wait()


_sc_scatter = functools.partial(
    pl.kernel,
    mesh=plsc.VectorSubcoreMesh(core_axis_name="c", subcore_axis_name="s",
                                num_cores=NC, num_subcores=NS),
    compiler_params=pltpu.CompilerParams(needs_layout_passes=False),
    out_type=jax.ShapeDtypeStruct((NC, 3 * R), jnp.float32),
    scratch_types=[
        [pltpu.VMEM((CH + 16,), jnp.int32)] * NB,
        [pltpu.VMEM((CH,), jnp.float32)] * NB,
        [pltpu.VMEM((CH,), jnp.float32)] * NB,
        [pltpu.VMEM((CH,), jnp.float32)] * NB,
        [pltpu.VMEM((CH,), jnp.float32)] * NB,
        [pltpu.VMEM((CP,), jnp.int32)] * NB,
        [pltpu.VMEM((CP,), jnp.float32)] * NB,
        [pltpu.VMEM((CP,), jnp.float32)] * NB,
        [pltpu.VMEM((CP,), jnp.float32)] * NB,
        pltpu.VMEM_SHARED((R,), jnp.float32),
        pltpu.VMEM_SHARED((R,), jnp.float32),
        pltpu.VMEM_SHARED((R,), jnp.float32),
        pltpu.SMEM((NB,), jnp.int32),
        [pltpu.SemaphoreType.DMA] * NB,
        [pltpu.SemaphoreType.DMA] * NB,
        pltpu.SemaphoreType.DMA,
    ],
)(_sc_body)


def _merge_body(p_ref, o_ref):
    o_ref[...] = (p_ref[0] + p_ref[1]).T


def kernel(ray_samples_packed, rgb_samples, weights_samples):
    zeros = jnp.zeros((R,), jnp.float32)
    partial = _sc_scatter(ray_samples_packed,
                          rgb_samples[:, 0], rgb_samples[:, 1],
                          rgb_samples[:, 2], weights_samples[:, 0], zeros)
    return pl.pallas_call(
        _merge_body,
        out_shape=jax.ShapeDtypeStruct((R, 3), jnp.float32),
    )(partial.reshape(NC, 3, R))


# speedup vs baseline: 1.3127x; 1.3127x over previous
import functools

import jax
import jax.numpy as jnp
from jax import lax
from jax.experimental import pallas as pl
from jax.experimental.pallas import tpu as pltpu
from jax.experimental.pallas import tpu_sc as plsc

N = 4194304
R = 65536
NC = 2
NS = 16
W = NC * NS
C = N // W
CH = 2048
NCH = C // CH
NJ = CH // 128
NB = 4
RT = R // NS


def _sc_body(ids_hbm, r_hbm, g_hbm, b_hbm, w_hbm, zeros_hbm, out_hbm,
             ids_v, r_v, g_v, b_v, w_v, cr_v, cg_v, cb_v,
             acc_r, acc_g, acc_b, sem_in, sem_sc, sem_z):
    cid = lax.axis_index("c")
    sid = lax.axis_index("s")
    wid = cid * NS + sid

    for acc in (acc_r, acc_g, acc_b):
        pltpu.async_copy(zeros_hbm.at[pl.ds(sid * RT, RT)],
                         acc.at[pl.ds(sid * RT, RT)], sem_z).wait()
    plsc.subcore_barrier()

    planes = ((r_hbm, r_v, cr_v, acc_r),
              (g_hbm, g_v, cg_v, acc_g),
              (b_hbm, b_v, cb_v, acc_b))

    def in_copies(p, ch):
        base = wid * C + ch * CH
        copies = [
            pltpu.make_async_copy(ids_hbm.at[pl.ds(base, CH)],
                                  ids_v[p], sem_in[p]),
        ]
        copies.append(pltpu.make_async_copy(w_hbm.at[pl.ds(base, CH)],
                                            w_v[p], sem_in[p]))
        copies.extend(
            pltpu.make_async_copy(x_hbm.at[pl.ds(base, CH)],
                                  x_v[p], sem_in[p])
            for x_hbm, x_v, _, _ in planes)
        return copies

    def issue_in(p, ch):
        for c in in_copies(p, ch):
            c.start()

    def wait_in(p, ch):
        for c in in_copies(p, ch):
            c.wait()

    def compute(p):
        def group(j, _):
            for gg in range(8):
                o = 128 * j + 16 * gg
                w16 = w_v[p][pl.ds(o, 16)]
                for _, x_v, c_v, _ in planes:
                    c_v[p][pl.ds(o, 16)] = x_v[p][pl.ds(o, 16)] * w16
            return 0

        lax.fori_loop(0, NJ, group, 0)

    def fire_scatters(p):
        for j in range(NJ):
            idx = ids_v[p].at[pl.ds(128 * j, 128)]
            for _, _, c_v, acc in planes:
                pltpu.async_copy(c_v[p].at[pl.ds(128 * j, 128)],
                                 acc.at[idx], sem_sc[p], add=True)

    def drain_scatters(p):
        for _ in range(3):
            pltpu.make_async_copy(w_hbm.at[pl.ds(0, CH)],
                                  w_v[p], sem_sc[p]).wait()

    issue_in(0, 0)
    issue_in(1, 1)

    def quad(t, _):
        for pp in range(NB):
            ch = NB * t + pp
            qq = (pp + 2) % NB

            def prefetch():
                issue_in(qq, ch + 2)

            def drain_and_prefetch():
                drain_scatters(qq)
                prefetch()

            if pp < 2:
                lax.cond(t > 0, drain_and_prefetch, prefetch)
            else:
                drain_scatters(qq)
                lax.cond(ch < NCH - 2, prefetch, lambda: None)
            wait_in(pp, ch)
            compute(pp)
            fire_scatters(pp)
        return 0

    lax.fori_loop(0, NCH // NB, quad, 0)
    drain_scatters((NCH - 2) % NB)
    drain_scatters((NCH - 1) % NB)
    plsc.subcore_barrier()

    for x, (_, _, _, acc) in enumerate(planes):
        pltpu.async_copy(acc.at[pl.ds(sid * RT, RT)],
                         out_hbm.at[cid].at[pl.ds(x * R + sid * RT, RT)],
                         sem_z).wait()


_sc_scatter = functools.partial(
    pl.kernel,
    mesh=plsc.VectorSubcoreMesh(core_axis_name="c", subcore_axis_name="s",
                                num_cores=NC, num_subcores=NS),
    compiler_params=pltpu.CompilerParams(needs_layout_passes=False),
    out_type=jax.ShapeDtypeStruct((NC, 3 * R), jnp.float32),
    scratch_types=[
        [pltpu.VMEM((CH,), jnp.int32)] * NB,
        [pltpu.VMEM((CH,), jnp.float32)] * NB,
        [pltpu.VMEM((CH,), jnp.float32)] * NB,
        [pltpu.VMEM((CH,), jnp.float32)] * NB,
        [pltpu.VMEM((CH,), jnp.float32)] * NB,
        [pltpu.VMEM((CH,), jnp.float32)] * NB,
        [pltpu.VMEM((CH,), jnp.float32)] * NB,
        [pltpu.VMEM((CH,), jnp.float32)] * NB,
        pltpu.VMEM_SHARED((R,), jnp.float32),
        pltpu.VMEM_SHARED((R,), jnp.float32),
        pltpu.VMEM_SHARED((R,), jnp.float32),
        [pltpu.SemaphoreType.DMA] * NB,
        [pltpu.SemaphoreType.DMA] * NB,
        pltpu.SemaphoreType.DMA,
    ],
)(_sc_body)


def _merge_body(p_ref, o_ref):
    o_ref[...] = (p_ref[0] + p_ref[1]).T


def kernel(ray_samples_packed, rgb_samples, weights_samples):
    zeros = jnp.zeros((R,), jnp.float32)
    partial = _sc_scatter(ray_samples_packed,
                          rgb_samples[:, 0], rgb_samples[:, 1],
                          rgb_samples[:, 2], weights_samples[:, 0], zeros)
    return pl.pallas_call(
        _merge_body,
        out_shape=jax.ShapeDtypeStruct((R, 3), jnp.float32),
    )(partial.reshape(NC, 3, R))


# speedup vs baseline: 1.4090x; 1.0734x over previous
import functools

import jax
import jax.numpy as jnp
from jax import lax
from jax.experimental import pallas as pl
from jax.experimental.pallas import tpu as pltpu
from jax.experimental.pallas import tpu_sc as plsc

N = 4194304
R = 65536
NC = 2
NS = 16
W = NC * NS
C = N // W
CH = 2048
NCH = C // CH
NJ = CH // 128
NB = 4
RT = R // NS


def _sc_body(ids_hbm, r_hbm, g_hbm, b_hbm, w_hbm, zeros_hbm, out_hbm,
             ids_v, r_v, g_v, b_v, w_v, cr_v, cg_v, cb_v,
             acc_r, acc_g, acc_b, sem_in, sem_sc, sem_z):
    cid = lax.axis_index("c")
    sid = lax.axis_index("s")
    wid = cid * NS + sid

    for acc in (acc_r, acc_g, acc_b):
        pltpu.async_copy(zeros_hbm.at[pl.ds(sid * RT, RT)],
                         acc.at[pl.ds(sid * RT, RT)], sem_z).wait()
    plsc.subcore_barrier()

    planes = ((r_hbm, r_v, cr_v, acc_r),
              (g_hbm, g_v, cg_v, acc_g),
              (b_hbm, b_v, cb_v, acc_b))

    def in_copies(p, ch):
        base = wid * C + ch * CH
        copies = [
            pltpu.make_async_copy(ids_hbm.at[pl.ds(base, CH)],
                                  ids_v[p], sem_in[p]),
        ]
        copies.append(pltpu.make_async_copy(w_hbm.at[pl.ds(base, CH)],
                                            w_v[p], sem_in[p]))
        copies.extend(
            pltpu.make_async_copy(x_hbm.at[pl.ds(base, CH)],
                                  x_v[p], sem_in[p])
            for x_hbm, x_v, _, _ in planes)
        return copies

    def issue_in(p, ch):
        for c in in_copies(p, ch):
            c.start()

    def wait_in(p, ch):
        for c in in_copies(p, ch):
            c.wait()

    def compute(p):
        def group(j, _):
            for gg in range(8):
                o = 128 * j + 16 * gg
                w16 = w_v[p][pl.ds(o, 16)]
                for _, x_v, c_v, _ in planes:
                    c_v[p][pl.ds(o, 16)] = x_v[p][pl.ds(o, 16)] * w16
            return 0

        lax.fori_loop(0, NJ, group, 0)

    def fire_scatters(p):
        for j in range(CH // 256):
            idx = ids_v[p].at[pl.ds(256 * j, 256)]
            for _, _, c_v, acc in planes:
                pltpu.async_copy(c_v[p].at[pl.ds(256 * j, 256)],
                                 acc.at[idx], sem_sc[p], add=True)

    def drain_scatters(p):
        for _ in range(3):
            pltpu.make_async_copy(w_hbm.at[pl.ds(0, CH)],
                                  w_v[p], sem_sc[p]).wait()

    issue_in(0, 0)
    issue_in(1, 1)

    def quad(t, _):
        for pp in range(NB):
            ch = NB * t + pp
            qq = (pp + 2) % NB

            def prefetch():
                issue_in(qq, ch + 2)

            def drain_and_prefetch():
                drain_scatters(qq)
                prefetch()

            if pp < 2:
                lax.cond(t > 0, drain_and_prefetch, prefetch)
            else:
                drain_scatters(qq)
                lax.cond(ch < NCH - 2, prefetch, lambda: None)
            wait_in(pp, ch)
            compute(pp)
            fire_scatters(pp)
        return 0

    lax.fori_loop(0, NCH // NB, quad, 0)
    drain_scatters((NCH - 2) % NB)
    drain_scatters((NCH - 1) % NB)
    plsc.subcore_barrier()

    for x, (_, _, _, acc) in enumerate(planes):
        pltpu.async_copy(acc.at[pl.ds(sid * RT, RT)],
                         out_hbm.at[cid].at[pl.ds(x * R + sid * RT, RT)],
                         sem_z).wait()


_sc_scatter = functools.partial(
    pl.kernel,
    mesh=plsc.VectorSubcoreMesh(core_axis_name="c", subcore_axis_name="s",
                                num_cores=NC, num_subcores=NS),
    compiler_params=pltpu.CompilerParams(needs_layout_passes=False),
    out_type=jax.ShapeDtypeStruct((NC, 3 * R), jnp.float32),
    scratch_types=[
        [pltpu.VMEM((CH,), jnp.int32)] * NB,
        [pltpu.VMEM((CH,), jnp.float32)] * NB,
        [pltpu.VMEM((CH,), jnp.float32)] * NB,
        [pltpu.VMEM((CH,), jnp.float32)] * NB,
        [pltpu.VMEM((CH,), jnp.float32)] * NB,
        [pltpu.VMEM((CH,), jnp.float32)] * NB,
        [pltpu.VMEM((CH,), jnp.float32)] * NB,
        [pltpu.VMEM((CH,), jnp.float32)] * NB,
        pltpu.VMEM_SHARED((R,), jnp.float32),
        pltpu.VMEM_SHARED((R,), jnp.float32),
        pltpu.VMEM_SHARED((R,), jnp.float32),
        [pltpu.SemaphoreType.DMA] * NB,
        [pltpu.SemaphoreType.DMA] * NB,
        pltpu.SemaphoreType.DMA,
    ],
)(_sc_body)


def _merge_body(p_ref, o_ref):
    o_ref[...] = (p_ref[0] + p_ref[1]).T


def kernel(ray_samples_packed, rgb_samples, weights_samples):
    zeros = jnp.zeros((R,), jnp.float32)
    partial = _sc_scatter(ray_samples_packed,
                          rgb_samples[:, 0], rgb_samples[:, 1],
                          rgb_samples[:, 2], weights_samples[:, 0], zeros)
    return pl.pallas_call(
        _merge_body,
        out_shape=jax.ShapeDtypeStruct((R, 3), jnp.float32),
    )(partial.reshape(NC, 3, R))


# speedup vs baseline: 1.4108x; 1.0013x over previous
import functools

import jax
import jax.numpy as jnp
from jax import lax
from jax.experimental import pallas as pl
from jax.experimental.pallas import tpu as pltpu
from jax.experimental.pallas import tpu_sc as plsc

N = 4194304
R = 65536
NC = 2
NS = 16
W = NC * NS
C = N // W
CH = 2048
NCH = C // CH
NJ = CH // 128
NB = 4
RT = R // NS


def _sc_body(ids_hbm, r_hbm, g_hbm, b_hbm, w_hbm, zeros_hbm, out_hbm,
             ids_v, r_v, g_v, b_v, w_v, cr_v, cg_v, cb_v,
             acc_r, acc_g, acc_b, sem_in, sem_sc, sem_z):
    cid = lax.axis_index("c")
    sid = lax.axis_index("s")
    wid = cid * NS + sid

    for acc in (acc_r, acc_g, acc_b):
        pltpu.async_copy(zeros_hbm.at[pl.ds(sid * RT, RT)],
                         acc.at[pl.ds(sid * RT, RT)], sem_z).wait()
    plsc.subcore_barrier()

    planes = ((r_hbm, r_v, cr_v, acc_r),
              (g_hbm, g_v, cg_v, acc_g),
              (b_hbm, b_v, cb_v, acc_b))

    def in_copies(p, ch):
        base = wid * C + ch * CH
        copies = [
            pltpu.make_async_copy(ids_hbm.at[pl.ds(base, CH)],
                                  ids_v[p], sem_in[p]),
        ]
        copies.append(pltpu.make_async_copy(w_hbm.at[pl.ds(base, CH)],
                                            w_v[p], sem_in[p]))
        copies.extend(
            pltpu.make_async_copy(x_hbm.at[pl.ds(base, CH)],
                                  x_v[p], sem_in[p])
            for x_hbm, x_v, _, _ in planes)
        return copies

    def issue_in(p, ch):
        for c in in_copies(p, ch):
            c.start()

    def wait_in(p, ch):
        for c in in_copies(p, ch):
            c.wait()

    def compute(p):
        def group(j, _):
            for gg in range(8):
                o = 128 * j + 16 * gg
                w16 = w_v[p][pl.ds(o, 16)]
                for _, x_v, c_v, _ in planes:
                    c_v[p][pl.ds(o, 16)] = x_v[p][pl.ds(o, 16)] * w16
            return 0

        lax.fori_loop(0, NJ, group, 0)

    def fire_scatters(p):
        idx = ids_v[p]
        for _, _, c_v, acc in planes:
            pltpu.async_copy(c_v[p], acc.at[idx], sem_sc[p], add=True)

    def drain_scatters(p):
        for _ in range(3):
            pltpu.make_async_copy(w_hbm.at[pl.ds(0, CH)],
                                  w_v[p], sem_sc[p]).wait()

    issue_in(0, 0)
    issue_in(1, 1)

    def quad(t, _):
        for pp in range(NB):
            ch = NB * t + pp
            qq = (pp + 2) % NB

            def prefetch():
                issue_in(qq, ch + 2)

            def drain_and_prefetch():
                drain_scatters(qq)
                prefetch()

            if pp < 2:
                lax.cond(t > 0, drain_and_prefetch, prefetch)
            else:
                drain_scatters(qq)
                lax.cond(ch < NCH - 2, prefetch, lambda: None)
            wait_in(pp, ch)
            compute(pp)
            fire_scatters(pp)
        return 0

    lax.fori_loop(0, NCH // NB, quad, 0)
    drain_scatters((NCH - 2) % NB)
    drain_scatters((NCH - 1) % NB)
    plsc.subcore_barrier()

    for x, (_, _, _, acc) in enumerate(planes):
        pltpu.async_copy(acc.at[pl.ds(sid * RT, RT)],
                         out_hbm.at[cid].at[pl.ds(x * R + sid * RT, RT)],
                         sem_z).wait()


_sc_scatter = functools.partial(
    pl.kernel,
    mesh=plsc.VectorSubcoreMesh(core_axis_name="c", subcore_axis_name="s",
                                num_cores=NC, num_subcores=NS),
    compiler_params=pltpu.CompilerParams(needs_layout_passes=False),
    out_type=jax.ShapeDtypeStruct((NC, 3 * R), jnp.float32),
    scratch_types=[
        [pltpu.VMEM((CH,), jnp.int32)] * NB,
        [pltpu.VMEM((CH,), jnp.float32)] * NB,
        [pltpu.VMEM((CH,), jnp.float32)] * NB,
        [pltpu.VMEM((CH,), jnp.float32)] * NB,
        [pltpu.VMEM((CH,), jnp.float32)] * NB,
        [pltpu.VMEM((CH,), jnp.float32)] * NB,
        [pltpu.VMEM((CH,), jnp.float32)] * NB,
        [pltpu.VMEM((CH,), jnp.float32)] * NB,
        pltpu.VMEM_SHARED((R,), jnp.float32),
        pltpu.VMEM_SHARED((R,), jnp.float32),
        pltpu.VMEM_SHARED((R,), jnp.float32),
        [pltpu.SemaphoreType.DMA] * NB,
        [pltpu.SemaphoreType.DMA] * NB,
        pltpu.SemaphoreType.DMA,
    ],
)(_sc_body)


def _merge_body(p_ref, o_ref):
    o_ref[...] = (p_ref[0] + p_ref[1]).T


def kernel(ray_samples_packed, rgb_samples, weights_samples):
    zeros = jnp.zeros((R,), jnp.float32)
    partial = _sc_scatter(ray_samples_packed,
                          rgb_samples[:, 0], rgb_samples[:, 1],
                          rgb_samples[:, 2], weights_samples[:, 0], zeros)
    return pl.pallas_call(
        _merge_body,
        out_shape=jax.ShapeDtypeStruct((R, 3), jnp.float32),
    )(partial.reshape(NC, 3, R))


# speedup vs baseline: 1.4150x; 1.0030x over previous
import functools

import jax
import jax.numpy as jnp
from jax import lax
from jax.experimental import pallas as pl
from jax.experimental.pallas import tpu as pltpu
from jax.experimental.pallas import tpu_sc as plsc

N = 4194304
R = 65536
NC = 2
NS = 16
W = NC * NS
C = N // W
CH = 2048
NCH = C // CH
NB = 4
RT = R // NS


def _sc_body(ids_hbm, r_hbm, g_hbm, b_hbm, w_hbm, zeros_hbm, out_hbm,
             ids_v, r_v, g_v, b_v, w_v, cr_v, cg_v, cb_v,
             acc_r, acc_g, acc_b, sem_in, sem_sc, sem_z):
    cid = lax.axis_index("c")
    sid = lax.axis_index("s")
    wid = cid * NS + sid

    for acc in (acc_r, acc_g, acc_b):
        pltpu.async_copy(zeros_hbm.at[pl.ds(sid * RT, RT)],
                         acc.at[pl.ds(sid * RT, RT)], sem_z).wait()
    plsc.subcore_barrier()

    planes = ((r_hbm, r_v, cr_v, acc_r),
              (g_hbm, g_v, cg_v, acc_g),
              (b_hbm, b_v, cb_v, acc_b))

    def in_copies(p, ch):
        base = wid * C + ch * CH
        copies = [
            pltpu.make_async_copy(ids_hbm.at[pl.ds(base, CH)],
                                  ids_v[p], sem_in[p]),
        ]
        copies.append(pltpu.make_async_copy(w_hbm.at[pl.ds(base, CH)],
                                            w_v[p], sem_in[p]))
        copies.extend(
            pltpu.make_async_copy(x_hbm.at[pl.ds(base, CH)],
                                  x_v[p], sem_in[p])
            for x_hbm, x_v, _, _ in planes)
        return copies

    def issue_in(p, ch):
        for c in in_copies(p, ch):
            c.start()

    def wait_in(p, ch):
        for c in in_copies(p, ch):
            c.wait()

    def compute(p):
        def group(j, _):
            for gg in range(8):
                o = 128 * j + 16 * gg
                w16 = w_v[p][pl.ds(o, 16)]
                for _, x_v, c_v, _ in planes:
                    c_v[p][pl.ds(o, 16)] = x_v[p][pl.ds(o, 16)] * w16
            return 0

        lax.fori_loop(0, CH // 128, group, 0)

    def fire_scatters(p):
        idx = ids_v[p]
        for _, _, c_v, acc in planes:
            pltpu.async_copy(c_v[p], acc.at[idx], sem_sc[p], add=True)

    def drain_scatters(p):
        for _ in range(3):
            pltpu.make_async_copy(w_hbm.at[pl.ds(0, CH)],
                                  w_v[p], sem_sc[p]).wait()

    issue_in(0, 0)
    issue_in(1, 1)

    def quad(t, _):
        for pp in range(NB):
            ch = NB * t + pp
            qq = (pp + 2) % NB

            def prefetch():
                issue_in(qq, ch + 2)

            def drain_and_prefetch():
                drain_scatters(qq)
                prefetch()

            if pp < 2:
                lax.cond(t > 0, drain_and_prefetch, prefetch)
            else:
                drain_scatters(qq)
                lax.cond(ch < NCH - 2, prefetch, lambda: None)
            wait_in(pp, ch)
            compute(pp)
            fire_scatters(pp)
        return 0

    lax.fori_loop(0, NCH // NB, quad, 0)
    drain_scatters((NCH - 2) % NB)
    drain_scatters((NCH - 1) % NB)
    plsc.subcore_barrier()

    for x, (_, _, _, acc) in enumerate(planes):
        pltpu.async_copy(acc.at[pl.ds(sid * RT, RT)],
                         out_hbm.at[cid].at[pl.ds(x * R + sid * RT, RT)],
                         sem_z).wait()


_sc_scatter = functools.partial(
    pl.kernel,
    mesh=plsc.VectorSubcoreMesh(core_axis_name="c", subcore_axis_name="s",
                                num_cores=NC, num_subcores=NS),
    compiler_params=pltpu.CompilerParams(needs_layout_passes=False),
    out_type=jax.ShapeDtypeStruct((NC, 3 * R), jnp.float32),
    scratch_types=[
        [pltpu.VMEM((CH,), jnp.int32)] * NB,
        [pltpu.VMEM((CH,), jnp.float32)] * NB,
        [pltpu.VMEM((CH,), jnp.float32)] * NB,
        [pltpu.VMEM((CH,), jnp.float32)] * NB,
        [pltpu.VMEM((CH,), jnp.float32)] * NB,
        [pltpu.VMEM((CH,), jnp.float32)] * NB,
        [pltpu.VMEM((CH,), jnp.float32)] * NB,
        [pltpu.VMEM((CH,), jnp.float32)] * NB,
        pltpu.VMEM_SHARED((R,), jnp.float32),
        pltpu.VMEM_SHARED((R,), jnp.float32),
        pltpu.VMEM_SHARED((R,), jnp.float32),
        [pltpu.SemaphoreType.DMA] * NB,
        [pltpu.SemaphoreType.DMA] * NB,
        pltpu.SemaphoreType.DMA,
    ],
)(_sc_body)


def _merge_body(p_ref, o_ref):
    o_ref[...] = (p_ref[0] + p_ref[1]).T


def kernel(ray_samples_packed, rgb_samples, weights_samples):
    zeros = jnp.zeros((R,), jnp.float32)
    partial = _sc_scatter(ray_samples_packed,
                          rgb_samples[:, 0], rgb_samples[:, 1],
                          rgb_samples[:, 2], weights_samples[:, 0], zeros)
    return pl.pallas_call(
        _merge_body,
        out_shape=jax.ShapeDtypeStruct((R, 3), jnp.float32),
    )(partial.reshape(NC, 3, R))
